# baseline (speedup 1.0000x reference)
import functools

import jax
import jax.numpy as jnp
from jax import lax
from jax.experimental import pallas as pl
from jax.experimental.pallas import tpu as pltpu
from jax.experimental.pallas import tpu_sc as plsc

B, N, H = 32, 2048, 512
L = 16
KH = H // L
CH = 64
NCH = N // CH
NEG = -1e30
LN2 = 0.6931471805599453

_MESH = plsc.VectorSubcoreMesh(core_axis_name="c", subcore_axis_name="s")


def _sc_body(hv, w, d, probs, logp, w_v, d_v, sc_v, lp_v, buf0, buf1, buf2,
             sem0, sem1, sem2, semw):
    wid = lax.axis_index("s") * 2 + lax.axis_index("c")
    iota = lax.broadcasted_iota(jnp.int32, (L,), 0)

    DIAG_DMA_ONLY = True

    def matvec_chunk(buf, base):
        if DIAG_DMA_ONLY:
            def g2(g, carry):
                sc_v[pl.ds(base + g * L, L)] = buf[0, pl.ds(0, L)]
                return carry
            lax.fori_loop(0, CH // L, g2, 0)
            return
        def group_body(g, carry):
            def row_body(j, vec):
                r = g * L + j
                accs = [buf[r, pl.ds(L * k, L)] * wch[k] for k in range(4)]
                for k in range(4, KH):
                    accs[k % 4] = accs[k % 4] + buf[r, pl.ds(L * k, L)] * wch[k]
                acc = (accs[0] + accs[1]) + (accs[2] + accs[3])
                return jnp.where(iota == j, jnp.sum(acc), vec)
            vec = lax.fori_loop(0, L, row_body, jnp.zeros((L,), jnp.float32),
                                unroll=1)
            sc_v[pl.ds(base + g * L, L)] = vec
            return carry
        lax.fori_loop(0, CH // L, group_body, 0)

    def start(chunk, buf, sem):
        pltpu.make_async_copy(hv.at[wid, pl.ds(chunk * CH, CH)], buf,
                              sem).start()

    def wait(chunk, buf, sem):
        pltpu.make_async_copy(hv.at[wid, pl.ds(chunk * CH, CH)], buf,
                              sem).wait()

    bufs = (buf0, buf1, buf2)
    sems = (sem0, sem1, sem2)
    for j in range(3):
        start(j, bufs[j], sems[j])
    pltpu.make_async_copy(w, w_v, semw).start()
    pltpu.make_async_copy(d, d_v, semw).start()
    pltpu.make_async_copy(w, w_v, semw).wait()
    pltpu.make_async_copy(d, d_v, semw).wait()
    wch = [w_v[pl.ds(L * k, L)] for k in range(KH)]

    def outer(i, carry):
        c0 = 3 * i
        for u in range(3):
            j = c0 + u
            s = u
            wait(j, bufs[s], sems[s])
            matvec_chunk(bufs[s], j * CH)
            start(lax.rem(j + 3, NCH), bufs[s], sems[s])
        return carry

    lax.fori_loop(0, NCH // 3, outer, 0)
    for j in range(NCH - NCH % 3, NCH):
        s = j % 3
        wait(j, bufs[s], sems[s])
        matvec_chunk(bufs[s], j * CH)
    for j in range(3 - NCH % 3):
        wait(0, bufs[(NCH + j) % 3], sems[(NCH + j) % 3])

    dvec = d_v[pl.ds(jnp.bitwise_and(wid, ~(L - 1)), L)]
    ddf = jnp.max(jnp.where(iota == jnp.bitwise_and(wid, L - 1),
                            dvec.astype(jnp.float32), 0.0))
    dd = ddf.astype(jnp.int32)
    svec = sc_v[pl.ds(jnp.bitwise_and(dd, ~(L - 1)), L)]
    sd = jnp.max(jnp.where(iota == jnp.bitwise_and(dd, L - 1), svec, NEG))
    last = sc_v[pl.ds(N - L, L)]
    sc_v[pl.ds(N - L, L)] = jnp.where(iota == L - 1, NEG, last)

    def mx_body(j, mv):
        return jnp.maximum(mv, sc_v[pl.ds(L * j, L)])

    mv = lax.fori_loop(0, N // L, mx_body, jnp.full((L,), NEG, jnp.float32))
    m = jnp.max(mv)

    def ex_body(j, sv):
        e = jnp.exp(sc_v[pl.ds(L * j, L)] - m)
        sc_v[pl.ds(L * j, L)] = e
        return sv + e

    sv = lax.fori_loop(0, N // L, ex_body, jnp.zeros((L,), jnp.float32))
    ssum = jnp.sum(sv)
    inv_v = jnp.full((L,), 1.0, jnp.float32) / jnp.full((L,), ssum)

    def nm_body(j, carry):
        sc_v[pl.ds(L * j, L)] = sc_v[pl.ds(L * j, L)] * inv_v
        return carry

    lax.fori_loop(0, N // L, nm_body, 0)
    pltpu.sync_copy(sc_v, probs.at[wid])

    vv = jnp.full((L,), 1.0, jnp.float32) * ssum
    bits = plsc.bitcast(vv, jnp.int32)
    y = bits.astype(jnp.float32) * (LN2 / 2.0**23) - 127.0 * LN2
    y = y + vv * jnp.exp(-y) - 1.0
    y = y + vv * jnp.exp(-y) - 1.0
    y = y + vv * jnp.exp(-y) - 1.0
    lp_v[...] = (sd - m) - y
    pltpu.sync_copy(lp_v, logp.at[wid])


_sc_call = functools.partial(
    pl.kernel,
    out_type=[
        jax.ShapeDtypeStruct((B, N), jnp.float32),
        jax.ShapeDtypeStruct((B, L), jnp.float32),
    ],
    mesh=_MESH,
    compiler_params=pltpu.CompilerParams(needs_layout_passes=False),
    scratch_types=[
        pltpu.VMEM((H,), jnp.float32),
        pltpu.VMEM((B,), jnp.int32),
        pltpu.VMEM((N,), jnp.float32),
        pltpu.VMEM((L,), jnp.float32),
        pltpu.VMEM((CH, H), jnp.float32),
        pltpu.VMEM((CH, H), jnp.float32),
        pltpu.VMEM((CH, H), jnp.float32),
        pltpu.SemaphoreType.DMA,
        pltpu.SemaphoreType.DMA,
        pltpu.SemaphoreType.DMA,
        pltpu.SemaphoreType.DMA,
    ],
)(_sc_body)


def kernel(hv, W, b, d):
    del b
    probs, logp = _sc_call(hv, W[:H], d.astype(jnp.int32))
    return probs[:, : N - 1], logp[:, :1]


# speedup vs baseline: 1.3986x; 1.3986x over previous
import functools

import jax
import jax.numpy as jnp
from jax import lax
from jax.experimental import pallas as pl
from jax.experimental.pallas import tpu as pltpu

B, N, H = 32, 2048, 512
NEG = -1e30


def _tc_body(d_ref, w_ref, hv_ref, probs_ref, logp_ref):
    g = pl.program_id(0)
    hv = hv_ref[0]
    w1 = w_ref[0:1, :]
    a = jnp.sum(hv * w1, axis=1)
    n_iota = lax.broadcasted_iota(jnp.int32, (N,), 0)
    valid = n_iota < (N - 1)
    s = jnp.where(valid, a, NEG)
    m = jnp.max(s)
    e = jnp.exp(s - m)
    ssum = jnp.sum(e)
    probs_ref[0, 0, :] = e / ssum
    dd = d_ref[g]
    logz = m + jnp.log(ssum)
    lp = jnp.sum(jnp.where(n_iota == dd, a, 0.0)) - logz
    logp_ref[0, 0, 0] = lp


def kernel(hv, W, b, d):
    del b
    w2d = W[:H].reshape(1, H)
    d32 = d.astype(jnp.int32)
    probs_p, logp = pl.pallas_call(
        _tc_body,
        grid_spec=pltpu.PrefetchScalarGridSpec(
            num_scalar_prefetch=1,
            grid=(B,),
            in_specs=[
                pl.BlockSpec((1, H), lambda g, dref: (0, 0)),
                pl.BlockSpec((1, N, H), lambda g, dref: (g, 0, 0)),
            ],
            out_specs=[
                pl.BlockSpec((1, 1, N), lambda g, dref: (g, 0, 0)),
                pl.BlockSpec((1, 1, 1), lambda g, dref: (g, 0, 0),
                             memory_space=pltpu.SMEM),
            ],
        ),
        out_shape=[
            jax.ShapeDtypeStruct((B, 1, N), jnp.float32),
            jax.ShapeDtypeStruct((B, 1, 1), jnp.float32),
        ],
    )(d32, w2d, hv)
    return probs_p[:, 0, : N - 1], logp[:, 0, :]
